# baseline (speedup 1.0000x reference)
import functools

import jax
import jax.numpy as jnp
import numpy as np
from jax import lax
from jax.experimental import pallas as pl
from jax.experimental.pallas import tpu as pltpu
from jax.experimental.pallas import tpu_sc as plsc

VOCAB = 100000
EMBED = 128
SEG_VOCAB = 3

NC = 2
NS = 16
NW = NC * NS
CHUNK = 80
NBUF = 4


def _sinusoidal_pe(seq_len, d_model):
    pos = jnp.arange(seq_len, dtype=jnp.float32)[:, None]
    div_term = jnp.exp(
        jnp.arange(0, d_model, 2, dtype=jnp.float32) * (-np.log(10000.0) / d_model)
    )
    pe = jnp.zeros((seq_len, d_model), dtype=jnp.float32)
    pe = pe.at[:, 0::2].set(jnp.sin(pos * div_term))
    pe = pe.at[:, 1::2].set(jnp.cos(pos * div_term))
    return pe


def _make_sc_kernel(n_rows, n_comb, seq_len):
    rows_per_w = n_rows // NW
    n_chunks = rows_per_w // CHUNK
    assert n_chunks % NBUF == 0
    assert rows_per_w % seq_len == 0
    mesh = plsc.VectorSubcoreMesh(core_axis_name="c", subcore_axis_name="s")

    @functools.partial(
        pl.kernel,
        mesh=mesh,
        out_type=jax.ShapeDtypeStruct((n_rows, EMBED), jnp.float32),
        scratch_types=[
            pltpu.VMEM((n_chunks, CHUNK), jnp.int32),
            pltpu.VMEM((rows_per_w,), jnp.int32),
            pltpu.VMEM((rows_per_w,), jnp.int32),
            pltpu.VMEM_SHARED((n_comb, EMBED), jnp.float32),
            [pltpu.VMEM((CHUNK, EMBED), jnp.float32)] * NBUF,
            [pltpu.VMEM((CHUNK, EMBED), jnp.float32)] * NBUF,
            [pltpu.SemaphoreType.DMA] * NBUF,
            [pltpu.SemaphoreType.DMA] * NBUF,
            [pltpu.SemaphoreType.DMA] * NBUF,
        ],
    )
    def k(tok_hbm, tidx_hbm, seg_hbm, comb_hbm, out_hbm,
          tidx_v, seg_v, cidx_v, comb_sh, tok_b, comb_b, sem_g, sem_c, sem_s):
        wid = lax.axis_index("s") * NC + lax.axis_index("c")
        base = wid * rows_per_w

        @pl.when(lax.axis_index("s") == 0)
        def _():
            pltpu.sync_copy(comb_hbm, comb_sh)

        pltpu.sync_copy(tidx_hbm.at[wid], tidx_v)

        def issue_tok(g, b):
            pltpu.async_copy(tok_hbm.at[tidx_v.at[g]], tok_b[b], sem_g[b])

        def issue_comb(g, b):
            pltpu.async_copy(comb_sh.at[cidx_v.at[pl.ds(g * CHUNK, CHUNK)]],
                             comb_b[b], sem_c[b])

        def wait_gathers(b):
            pltpu.make_async_copy(tok_hbm.at[pl.ds(0, CHUNK)], tok_b[b],
                                  sem_g[b]).wait()
            pltpu.make_async_copy(comb_sh.at[pl.ds(0, CHUNK)], comb_b[b],
                                  sem_c[b]).wait()

        def wait_scatter(b):
            pltpu.make_async_copy(tok_b[b], out_hbm.at[pl.ds(0, CHUNK)],
                                  sem_s[b]).wait()

        issue_tok(0, 0)
        issue_tok(1, 1)

        pltpu.sync_copy(seg_hbm.at[wid], seg_v)
        lane = lax.iota(jnp.int32, 16)

        def cidx_body(i, carry):
            pos = (lane + i * 16) % seq_len
            cidx_v[pl.ds(i * 16, 16)] = pos * SEG_VOCAB + seg_v[pl.ds(i * 16, 16)]
            return carry

        lax.fori_loop(0, rows_per_w // 16, cidx_body, 0, unroll=8)
        plsc.subcore_barrier()

        issue_comb(0, 0)
        issue_comb(1, 1)

        def outer(g0, carry):
            g0 = g0 * NBUF
            for b in range(NBUF):
                g = g0 + b
                bn = (b + 2) % NBUF
                @pl.when(g >= 2)
                def _():
                    wait_scatter(bn)

                @pl.when(g + 2 < n_chunks)
                def _():
                    issue_tok(g + 2, bn)
                    issue_comb(g + 2, bn)

                wait_gathers(b)

                def row_body(r, c2):
                    for j in range(EMBED // 16):
                        plsc.addupdate(
                            tok_b[b].at[r, pl.ds(j * 16, 16)],
                            comb_b[b][r, pl.ds(j * 16, 16)],
                        )
                    return c2

                lax.fori_loop(0, CHUNK, row_body, 0, unroll=8)
                pltpu.async_copy(
                    tok_b[b], out_hbm.at[pl.ds(base + g * CHUNK, CHUNK)], sem_s[b])
            return carry

        lax.fori_loop(0, n_chunks // NBUF, outer, 0, unroll=False)
        wait_scatter((n_chunks - 2) % NBUF)
        wait_scatter((n_chunks - 1) % NBUF)

    return k


def kernel(sequence, segment_label, token_table, seg_table):
    B, L = sequence.shape
    d_model = token_table.shape[1]
    n_rows = B * L
    n_comb = L * SEG_VOCAB

    pe = _sinusoidal_pe(L, d_model)
    comb = (pe[:, None, :] + seg_table[None, :, :]).reshape(n_comb, d_model)

    rows_per_w = n_rows // NW
    tidx = sequence.astype(jnp.int32).reshape(NW, rows_per_w // CHUNK, CHUNK)
    seg = segment_label.astype(jnp.int32).reshape(NW, rows_per_w)

    out = _make_sc_kernel(n_rows, n_comb, L)(token_table, tidx, seg, comb)
    return out.reshape(B, L, d_model)


# speedup vs baseline: 1.0259x; 1.0259x over previous
import functools

import jax
import jax.numpy as jnp
import numpy as np
from jax import lax
from jax.experimental import pallas as pl
from jax.experimental.pallas import tpu as pltpu
from jax.experimental.pallas import tpu_sc as plsc

VOCAB = 100000
EMBED = 128
SEG_VOCAB = 3

NC = 2
NS = 16
NW = NC * NS
CHUNK = 64
NBUF = 5


def _sinusoidal_pe(seq_len, d_model):
    pos = np.arange(seq_len, dtype=np.float32)[:, None]
    div_term = np.exp(
        np.arange(0, d_model, 2, dtype=np.float32)
        * np.float32(-np.log(10000.0) / d_model)
    ).astype(np.float32)
    pe = np.zeros((seq_len, d_model), dtype=np.float32)
    pe[:, 0::2] = np.sin((pos * div_term).astype(np.float32)).astype(np.float32)
    pe[:, 1::2] = np.cos((pos * div_term).astype(np.float32)).astype(np.float32)
    return jnp.asarray(pe)


def _make_sc_kernel(n_rows, n_comb, seq_len):
    rows_per_w = n_rows // NW
    n_chunks = rows_per_w // CHUNK
    assert n_chunks % NBUF == 0
    assert rows_per_w % seq_len == 0
    mesh = plsc.VectorSubcoreMesh(core_axis_name="c", subcore_axis_name="s")

    @functools.partial(
        pl.kernel,
        mesh=mesh,
        out_type=jax.ShapeDtypeStruct((n_rows, EMBED), jnp.float32),
        scratch_types=[
            pltpu.VMEM((rows_per_w,), jnp.int32),
            pltpu.VMEM((rows_per_w,), jnp.int32),
            pltpu.VMEM((rows_per_w,), jnp.int32),
            pltpu.VMEM_SHARED((n_comb, EMBED), jnp.float32),
            [pltpu.VMEM((CHUNK, EMBED), jnp.float32)] * NBUF,
            [pltpu.VMEM((CHUNK, EMBED), jnp.float32)] * NBUF,
            [pltpu.SemaphoreType.DMA] * NBUF,
            [pltpu.SemaphoreType.DMA] * NBUF,
            [pltpu.SemaphoreType.DMA] * NBUF,
        ],
    )
    def k(tok_hbm, pack_hbm, comb_hbm, out_hbm,
          pack_v, tidx_v, cidx_v, comb_sh, tok_b, comb_b, sem_g, sem_c, sem_s):
        wid = lax.axis_index("s") * NC + lax.axis_index("c")
        base = wid * rows_per_w

        @pl.when(lax.axis_index("s") == 0)
        def _():
            pltpu.sync_copy(comb_hbm, comb_sh)

        pltpu.sync_copy(pack_hbm.at[pl.ds(base, rows_per_w)], pack_v)
        lane = lax.iota(jnp.int32, 16)

        def unpack_chunk(c):
            for grp in range(CHUNK // 16):
                off = c * CHUNK + grp * 16
                p = pack_v[pl.ds(off, 16)]
                tidx_v[pl.ds(off, 16)] = p & 0x1FFFF
                pos = (lane + off) % seq_len
                cidx_v[pl.ds(off, 16)] = pos * SEG_VOCAB + (
                    lax.shift_right_logical(p, 17))

        lax.fori_loop(0, 7, lambda c, _: (unpack_chunk(c), 0)[1], 0,
                      unroll=False)

        def issue_tok(g, b):
            pltpu.async_copy(tok_hbm.at[tidx_v.at[pl.ds(g * CHUNK, CHUNK)]],
                             tok_b[b], sem_g[b])

        def issue_comb(g, b):
            pltpu.async_copy(comb_sh.at[cidx_v.at[pl.ds(g * CHUNK, CHUNK)]],
                             comb_b[b], sem_c[b])

        def wait_gathers(b):
            pltpu.make_async_copy(tok_hbm.at[pl.ds(0, CHUNK)], tok_b[b],
                                  sem_g[b]).wait()
            pltpu.make_async_copy(comb_sh.at[pl.ds(0, CHUNK)], comb_b[b],
                                  sem_c[b]).wait()

        def wait_scatter(b):
            pltpu.make_async_copy(tok_b[b], out_hbm.at[pl.ds(0, CHUNK)],
                                  sem_s[b]).wait()

        issue_tok(0, 0)
        issue_tok(1, 1)
        issue_tok(2, 2)

        plsc.subcore_barrier()

        issue_comb(0, 0)
        issue_comb(1, 1)
        issue_comb(2, 2)

        def outer(g0, carry):
            g0 = g0 * NBUF
            for b in range(NBUF):
                g = g0 + b
                bn = (b + 3) % NBUF
                @pl.when(g >= 2)
                def _():
                    wait_scatter(bn)

                @pl.when(g + 7 < n_chunks)
                def _():
                    unpack_chunk(g + 7)

                @pl.when(g + 3 < n_chunks)
                def _():
                    issue_tok(g + 3, bn)
                    issue_comb(g + 3, bn)

                wait_gathers(b)

                def row_body(r, c2):
                    for j in range(EMBED // 16):
                        plsc.addupdate(
                            tok_b[b].at[r, pl.ds(j * 16, 16)],
                            comb_b[b][r, pl.ds(j * 16, 16)],
                        )
                    return c2

                lax.fori_loop(0, CHUNK, row_body, 0, unroll=8)
                pltpu.async_copy(
                    tok_b[b], out_hbm.at[pl.ds(base + g * CHUNK, CHUNK)], sem_s[b])
            return carry

        lax.fori_loop(0, n_chunks // NBUF, outer, 0, unroll=False)
        wait_scatter((n_chunks - 2) % NBUF)
        wait_scatter((n_chunks - 1) % NBUF)

    return k


def kernel(sequence, segment_label, token_table, seg_table):
    B, L = sequence.shape
    d_model = token_table.shape[1]
    n_rows = B * L
    n_comb = L * SEG_VOCAB

    pe = _sinusoidal_pe(L, d_model)
    comb = (pe[:, None, :] + seg_table[None, :, :]).reshape(n_comb, d_model)

    pack = (sequence.astype(jnp.int32)
            | (segment_label.astype(jnp.int32) << 17)).reshape(n_rows)

    out = _make_sc_kernel(n_rows, n_comb, L)(token_table, pack, comb)
    return out.reshape(B, L, d_model)


# speedup vs baseline: 1.0720x; 1.0450x over previous
import functools

import jax
import jax.numpy as jnp
import numpy as np
from jax import lax
from jax.experimental import pallas as pl
from jax.experimental.pallas import tpu as pltpu
from jax.experimental.pallas import tpu_sc as plsc

VOCAB = 100000
EMBED = 128
SEG_VOCAB = 3

NC = 2
NS = 16
NW = NC * NS
CHUNK = 64
NBUF = 5


def _sinusoidal_pe(seq_len, d_model):
    pos = np.arange(seq_len, dtype=np.float32)[:, None]
    div_term = np.exp(
        np.arange(0, d_model, 2, dtype=np.float32)
        * np.float32(-np.log(10000.0) / d_model)
    ).astype(np.float32)
    pe = np.zeros((seq_len, d_model), dtype=np.float32)
    pe[:, 0::2] = np.sin((pos * div_term).astype(np.float32)).astype(np.float32)
    pe[:, 1::2] = np.cos((pos * div_term).astype(np.float32)).astype(np.float32)
    return jnp.asarray(pe)


def _make_sc_kernel(n_rows, n_comb, seq_len):
    rows_per_w = n_rows // NW
    n_chunks = rows_per_w // CHUNK
    assert n_chunks % NBUF == 0
    assert rows_per_w % seq_len == 0
    mesh = plsc.VectorSubcoreMesh(core_axis_name="c", subcore_axis_name="s")

    @functools.partial(
        pl.kernel,
        mesh=mesh,
        out_type=jax.ShapeDtypeStruct((n_rows, EMBED), jnp.float32),
        scratch_types=[
            pltpu.VMEM((rows_per_w,), jnp.int32),
            pltpu.VMEM((rows_per_w,), jnp.int32),
            pltpu.VMEM((rows_per_w,), jnp.int32),
            pltpu.VMEM_SHARED((n_comb, EMBED), jnp.float32),
            [pltpu.VMEM((CHUNK, EMBED), jnp.float32)] * NBUF,
            [pltpu.VMEM((CHUNK, EMBED), jnp.float32)] * NBUF,
            [pltpu.SemaphoreType.DMA] * NBUF,
            [pltpu.SemaphoreType.DMA] * NBUF,
            [pltpu.SemaphoreType.DMA] * NBUF,
        ],
    )
    def k(tok_hbm, pack_hbm, comb_hbm, out_hbm,
          pack_v, tidx_v, cidx_v, comb_sh, tok_b, comb_b, sem_g, sem_c, sem_s):
        wid = lax.axis_index("s") * NC + lax.axis_index("c")
        base = wid * rows_per_w

        @pl.when(lax.axis_index("s") == 0)
        def _():
            pltpu.sync_copy(comb_hbm, comb_sh)

        pltpu.sync_copy(pack_hbm.at[pl.ds(base, rows_per_w)], pack_v)
        lane = lax.iota(jnp.int32, 16)

        def unpack_body(i, carry):
            p = pack_v[pl.ds(i * 16, 16)]
            tidx_v[pl.ds(i * 16, 16)] = p & 0x1FFFF
            pos = (lane + i * 16) % seq_len
            cidx_v[pl.ds(i * 16, 16)] = pos * SEG_VOCAB + (
                lax.shift_right_logical(p, 17))
            return carry

        lax.fori_loop(0, rows_per_w // 16, unpack_body, 0, unroll=8)

        def issue_tok(g, b):
            pltpu.async_copy(tok_hbm.at[tidx_v.at[pl.ds(g * CHUNK, CHUNK)]],
                             tok_b[b], sem_g[b])

        def issue_comb(g, b):
            pltpu.async_copy(comb_sh.at[cidx_v.at[pl.ds(g * CHUNK, CHUNK)]],
                             comb_b[b], sem_c[b])

        def wait_gathers(b):
            pltpu.make_async_copy(tok_hbm.at[pl.ds(0, CHUNK)], tok_b[b],
                                  sem_g[b]).wait()
            pltpu.make_async_copy(comb_sh.at[pl.ds(0, CHUNK)], comb_b[b],
                                  sem_c[b]).wait()

        def wait_scatter(b):
            pltpu.make_async_copy(tok_b[b], out_hbm.at[pl.ds(0, CHUNK)],
                                  sem_s[b]).wait()

        issue_tok(0, 0)
        issue_tok(1, 1)
        issue_tok(2, 2)

        plsc.subcore_barrier()

        issue_comb(0, 0)
        issue_comb(1, 1)
        issue_comb(2, 2)

        def outer(g0, carry):
            g0 = g0 * NBUF
            for b in range(NBUF):
                g = g0 + b
                bn = (b + 3) % NBUF
                @pl.when(g >= 2)
                def _():
                    wait_scatter(bn)

                @pl.when(g + 3 < n_chunks)
                def _():
                    issue_tok(g + 3, bn)
                    issue_comb(g + 3, bn)

                wait_gathers(b)

                def row_body(r, c2):
                    for j in range(EMBED // 16):
                        plsc.addupdate(
                            tok_b[b].at[r, pl.ds(j * 16, 16)],
                            comb_b[b][r, pl.ds(j * 16, 16)],
                        )
                    return c2

                lax.fori_loop(0, CHUNK, row_body, 0, unroll=8)
                pltpu.async_copy(
                    tok_b[b], out_hbm.at[pl.ds(base + g * CHUNK, CHUNK)], sem_s[b])
            return carry

        lax.fori_loop(0, n_chunks // NBUF, outer, 0, unroll=False)
        wait_scatter((n_chunks - 2) % NBUF)
        wait_scatter((n_chunks - 1) % NBUF)

    return k


def kernel(sequence, segment_label, token_table, seg_table):
    B, L = sequence.shape
    d_model = token_table.shape[1]
    n_rows = B * L
    n_comb = L * SEG_VOCAB

    pe = _sinusoidal_pe(L, d_model)
    comb = (pe[:, None, :] + seg_table[None, :, :]).reshape(n_comb, d_model)

    pack = (sequence.astype(jnp.int32)
            | (segment_label.astype(jnp.int32) << 17)).reshape(n_rows)

    out = _make_sc_kernel(n_rows, n_comb, L)(token_table, pack, comb)
    return out.reshape(B, L, d_model)


# speedup vs baseline: 1.0740x; 1.0018x over previous
import functools

import jax
import jax.numpy as jnp
import numpy as np
from jax import lax
from jax.experimental import pallas as pl
from jax.experimental.pallas import tpu as pltpu
from jax.experimental.pallas import tpu_sc as plsc

VOCAB = 100000
EMBED = 128
SEG_VOCAB = 3

NC = 2
NS = 16
NW = NC * NS
CHUNK = 64
NBUF = 5


def _sinusoidal_pe(seq_len, d_model):
    pos = np.arange(seq_len, dtype=np.float32)[:, None]
    div_term = np.exp(
        np.arange(0, d_model, 2, dtype=np.float32)
        * np.float32(-np.log(10000.0) / d_model)
    ).astype(np.float32)
    pe = np.zeros((seq_len, d_model), dtype=np.float32)
    pe[:, 0::2] = np.sin((pos * div_term).astype(np.float32)).astype(np.float32)
    pe[:, 1::2] = np.cos((pos * div_term).astype(np.float32)).astype(np.float32)
    return jnp.asarray(pe)


def _make_sc_kernel(n_rows, n_comb, seq_len, shift):
    rows_per_w = n_rows // NW
    n_chunks = rows_per_w // CHUNK
    assert n_chunks % NBUF == 0
    assert rows_per_w % seq_len == 0
    mesh = plsc.VectorSubcoreMesh(core_axis_name="c", subcore_axis_name="s")

    @functools.partial(
        pl.kernel,
        mesh=mesh,
        out_type=jax.ShapeDtypeStruct((n_rows, EMBED), jnp.float32),
        scratch_types=[
            pltpu.VMEM((rows_per_w,), jnp.int32),
            pltpu.VMEM((rows_per_w,), jnp.int32),
            pltpu.VMEM((rows_per_w,), jnp.int32),
            pltpu.VMEM_SHARED((n_comb, EMBED), jnp.float32),
            [pltpu.VMEM((CHUNK, EMBED), jnp.float32)] * NBUF,
            [pltpu.VMEM((CHUNK, EMBED), jnp.float32)] * NBUF,
            [pltpu.SemaphoreType.DMA] * NBUF,
            [pltpu.SemaphoreType.DMA] * NBUF,
            [pltpu.SemaphoreType.DMA] * NBUF,
        ],
    )
    def k(tok_hbm, pack_hbm, comb_hbm, out_hbm,
          pack_v, tidx_v, cidx_v, comb_sh, tok_b, comb_b, sem_g, sem_c, sem_s):
        wid = lax.axis_index("s") * NC + lax.axis_index("c")
        base = wid * rows_per_w

        @pl.when(lax.axis_index("s") == 0)
        def _():
            pltpu.sync_copy(comb_hbm, comb_sh)

        pltpu.sync_copy(pack_hbm.at[pl.ds(base, rows_per_w)], pack_v)
        lane = lax.iota(jnp.int32, 16)

        def unpack_body(i, carry):
            p = pack_v[pl.ds(i * 16, 16)]
            tidx_v[pl.ds(i * 16, 16)] = p & ((1 << shift) - 1)
            pos = (lane + i * 16) % seq_len
            cidx_v[pl.ds(i * 16, 16)] = pos * SEG_VOCAB + (
                lax.shift_right_logical(p, shift))
            return carry

        lax.fori_loop(0, rows_per_w // 16, unpack_body, 0, unroll=8)

        def issue_tok(g, b):
            pltpu.async_copy(tok_hbm.at[tidx_v.at[pl.ds(g * CHUNK, CHUNK)]],
                             tok_b[b], sem_g[b])

        def issue_comb(g, b):
            pltpu.async_copy(comb_sh.at[cidx_v.at[pl.ds(g * CHUNK, CHUNK)]],
                             comb_b[b], sem_c[b])

        def wait_gathers(b):
            pltpu.make_async_copy(tok_hbm.at[pl.ds(0, CHUNK)], tok_b[b],
                                  sem_g[b]).wait()
            pltpu.make_async_copy(comb_sh.at[pl.ds(0, CHUNK)], comb_b[b],
                                  sem_c[b]).wait()

        def wait_scatter(b):
            pltpu.make_async_copy(tok_b[b], out_hbm.at[pl.ds(0, CHUNK)],
                                  sem_s[b]).wait()

        issue_tok(0, 0)
        issue_tok(1, 1)
        issue_tok(2, 2)

        plsc.subcore_barrier()

        issue_comb(0, 0)
        issue_comb(1, 1)
        issue_comb(2, 2)

        def outer(g0, carry):
            g0 = g0 * NBUF
            for b in range(NBUF):
                g = g0 + b
                bn = (b + 3) % NBUF
                @pl.when(g >= 2)
                def _():
                    wait_scatter(bn)

                @pl.when(g + 3 < n_chunks)
                def _():
                    issue_tok(g + 3, bn)
                    issue_comb(g + 3, bn)

                wait_gathers(b)

                def row_body(r, c2):
                    for j in range(EMBED // 16):
                        plsc.addupdate(
                            tok_b[b].at[r, pl.ds(j * 16, 16)],
                            comb_b[b][r, pl.ds(j * 16, 16)],
                        )
                    return c2

                lax.fori_loop(0, CHUNK, row_body, 0, unroll=8)
                pltpu.async_copy(
                    tok_b[b], out_hbm.at[pl.ds(base + g * CHUNK, CHUNK)], sem_s[b])
            return carry

        lax.fori_loop(0, n_chunks // NBUF, outer, 0, unroll=False)
        wait_scatter((n_chunks - 2) % NBUF)
        wait_scatter((n_chunks - 1) % NBUF)

    return k


def kernel(sequence, segment_label, token_table, seg_table):
    B, L = sequence.shape
    d_model = token_table.shape[1]
    n_rows = B * L
    n_comb = L * SEG_VOCAB

    pe = _sinusoidal_pe(L, d_model)
    comb = (pe[:, None, :] + seg_table[None, :, :]).reshape(n_comb, d_model)

    shift = max(1, int(token_table.shape[0] - 1).bit_length())
    pack = (sequence.astype(jnp.int32)
            | (segment_label.astype(jnp.int32) << shift)).reshape(n_rows)

    out = _make_sc_kernel(n_rows, n_comb, L, shift)(token_table, pack, comb)
    return out.reshape(B, L, d_model)
